# Initial kernel scaffold; baseline (speedup 1.0000x reference)
import jax
import jax.numpy as jnp
from jax.experimental import pallas as pl


def kernel(numerical_data, categorical_data, emb_tables, W1, b1, g1, be1, W2, b2, g2, be2, Wp, bp):
    raise NotImplementedError("write your pallas kernel here")


# speedup vs baseline: 7.9819x; 7.9819x over previous
import functools

import jax
import jax.numpy as jnp
from jax import lax
from jax.experimental import pallas as pl
from jax.experimental.pallas import tpu as pltpu
from jax.experimental.pallas import tpu_sc as plsc

B = 16384
NUM = 13
F = 26
V = 100000
D = 32

NC = 2
NS = 16
NW = NC * NS

BF = B * F
PER_W = BF // NW
CHUNK = 13 * 128
NCHUNK = PER_W // CHUNK
KROWS = CHUNK // 128

assert PER_W * NW == BF and NCHUNK * CHUNK == PER_W


def _sc_gather(table_flat, idx):
    mesh = plsc.VectorSubcoreMesh(core_axis_name="c", subcore_axis_name="s")

    @functools.partial(
        pl.kernel,
        out_type=jax.ShapeDtypeStruct((NW * NCHUNK, CHUNK, D), jnp.float32),
        mesh=mesh,
        scratch_types=[
            pltpu.VMEM((KROWS, 128), jnp.int32),
            pltpu.VMEM((CHUNK, D), jnp.float32),
            pltpu.SemaphoreType.DMA,
        ],
        compiler_params=pltpu.CompilerParams(use_tc_tiling_on_sc=False),
    )
    def gather_kernel(table_hbm, idx_hbm, out_hbm, idx_v, rows_v, sem):
        wid = lax.axis_index("s") * NC + lax.axis_index("c")

        def body(s, _):
            pltpu.sync_copy(idx_hbm.at[wid, s], idx_v)
            copies = []
            for j in range(KROWS):
                copies.append(
                    pltpu.async_copy(
                        table_hbm.at[idx_v.at[j]],
                        rows_v.at[pl.ds(j * 128, 128)],
                        sem,
                    )
                )
            for c in copies:
                c.wait()
            pltpu.sync_copy(rows_v, out_hbm.at[wid * NCHUNK + s])
            return _

        lax.fori_loop(0, NCHUNK, body, None)

    return gather_kernel(table_flat, idx)


_BM = 1024
_INV_SQRT = float(1.0 / (1.0 + 1e-5) ** 0.5)


def _mlp_kernel(num_ref, emb_ref, w1n_ref, w1e_ref, b1_ref, g1_ref, be1_ref,
                w2_ref, b2_ref, g2_ref, be2_ref, wp_ref, bp_ref, out_ref):
    x = jnp.dot(num_ref[...], w1n_ref[...], preferred_element_type=jnp.float32)
    x = x + jnp.dot(emb_ref[...], w1e_ref[...], preferred_element_type=jnp.float32)
    x = (x + b1_ref[...]) * (g1_ref[...] * _INV_SQRT) + be1_ref[...]
    x = jnp.maximum(x, 0.0)
    x = jnp.dot(x, w2_ref[...], preferred_element_type=jnp.float32)
    x = (x + b2_ref[...]) * (g2_ref[...] * _INV_SQRT) + be2_ref[...]
    x = jnp.maximum(x, 0.0)
    x = jnp.dot(x, wp_ref[...], preferred_element_type=jnp.float32)
    out_ref[...] = x + bp_ref[...]


def _mlp(numerical, emb, W1, b1, g1, be1, W2, b2, g2, be2, Wp, bp):
    W1n = W1[:NUM]
    W1e = W1[NUM:]
    row = lambda v: v.reshape(1, -1)
    grid = (B // _BM,)
    full = lambda shape: pl.BlockSpec(shape, lambda i: (0, 0))
    return pl.pallas_call(
        _mlp_kernel,
        grid=grid,
        in_specs=[
            pl.BlockSpec((_BM, NUM), lambda i: (i, 0)),
            pl.BlockSpec((_BM, F * D), lambda i: (i, 0)),
            full((NUM, 512)),
            full((F * D, 512)),
            full((1, 512)), full((1, 512)), full((1, 512)),
            full((512, 256)),
            full((1, 256)), full((1, 256)), full((1, 256)),
            full((256, 768)),
            full((1, 768)),
        ],
        out_specs=pl.BlockSpec((_BM, 768), lambda i: (i, 0)),
        out_shape=jax.ShapeDtypeStruct((B, 768), jnp.float32),
    )(numerical, emb, W1n, W1e, row(b1), row(g1), row(be1),
      W2, row(b2), row(g2), row(be2), Wp, row(bp))


def kernel(numerical_data, categorical_data, emb_tables, W1, b1, g1, be1,
           W2, b2, g2, be2, Wp, bp):
    table_flat = emb_tables.reshape(F * V, D)
    idx = (categorical_data.astype(jnp.int32)
           + (jnp.arange(F, dtype=jnp.int32) * V)[None, :])
    idx = idx.reshape(NW, NCHUNK, KROWS, 128)
    rows = _sc_gather(table_flat, idx)
    emb = rows.reshape(B, F * D)
    return _mlp(numerical_data, emb, W1, b1, g1, be1, W2, b2, g2, be2, Wp, bp)
